# baseline (speedup 1.0000x reference)
import functools

import jax
import jax.numpy as jnp
from jax import lax
from jax.experimental import pallas as pl
from jax.experimental.pallas import tpu as pltpu
from jax.experimental.pallas import tpu_sc as plsc

N_NODES = 10000
N_EDGES = 320000
D_FEAT = 128
TIME_WINDOW = 500

_B = 128
_TILES = 32
_NSLOT = 80
_EPAD = (_NSLOT + 1) * _TILES * _B
_NROWS = 10240
_ZROWS = _NROWS // 16


def _sc_body(x_hbm, src_hbm, dst_hbm, et_hbm, st_hbm, p_out, c_out,
             acc, accc,
             srcv0, srcv1, dstv0, dstv1, etv0, etv1, stv0, stv1,
             deff0, deff1, rows0, rows1, onesv, zb2, zb1,
             s_idx0, s_idx1, s_st0, s_st1, s_rows0, s_rows1):
    cid = lax.axis_index("c")
    sid = lax.axis_index("s")
    wid = sid * 2 + cid

    z16 = jnp.zeros((16,), jnp.float32)
    for i in range(16):
        for j in range(8):
            zb2[i, pl.ds(j * 16, 16)] = z16
    for k in range(_ZROWS // 16):
        zb1[pl.ds(k * 16, 16)] = z16
    for j in range(8):
        onesv[pl.ds(j * 16, 16)] = jnp.ones((16,), jnp.float32)

    def zloop(k, carry):
        pltpu.sync_copy(zb2, acc.at[pl.ds(sid * _ZROWS + k * 16, 16)])
        return carry

    lax.fori_loop(0, _ZROWS // 16, zloop, None)
    pltpu.sync_copy(zb1, accc.at[pl.ds(sid * _ZROWS, _ZROWS)])

    plsc.subcore_barrier()

    bufs = [
        (srcv0, dstv0, etv0, stv0, deff0, rows0, s_idx0, s_st0, s_rows0),
        (srcv1, dstv1, etv1, stv1, deff1, rows1, s_idx1, s_st1, s_rows1),
    ]

    def load_idx(g, b):
        srcv, dstv, etv = bufs[b][0], bufs[b][1], bufs[b][2]
        off = (g * _TILES + wid) * _B
        pltpu.sync_copy(src_hbm.at[pl.ds(off, _B)], srcv)
        pltpu.sync_copy(dst_hbm.at[pl.ds(off, _B)], dstv)
        pltpu.sync_copy(et_hbm.at[pl.ds(off, _B)], etv)

    def fire_rows(b):
        srcv, rows, s_rows = bufs[b][0], bufs[b][5], bufs[b][8]
        pltpu.make_async_copy(x_hbm.at[srcv], rows, s_rows).start()

    def wait_rows(b):
        srcv, rows, s_rows = bufs[b][0], bufs[b][5], bufs[b][8]
        pltpu.make_async_copy(x_hbm.at[srcv], rows, s_rows).wait()

    def fire_rows(b):
        srcv, rows, s_rows = bufs[b][0], bufs[b][5], bufs[b][8]
        pltpu.make_async_copy(x_hbm.at[srcv], rows, s_rows).start()

    def wait_rows(b):
        srcv, rows, s_rows = bufs[b][0], bufs[b][5], bufs[b][8]
        pltpu.make_async_copy(x_hbm.at[srcv], rows, s_rows).wait()

    def do_slot(g, b):
        srcv, dstv, etv, stv, deff, rows, s_idx, s_st, s_rows = bufs[b]
        load_idx(g + 1, b ^ 1)
        pltpu.async_copy(st_hbm.at[dstv], stv, s_st).wait()
        for j in range(_B // 16):
            sl = pl.ds(j * 16, 16)
            et = etv[sl]
            st = stv[sl]
            m = (et <= st) & (et > st - TIME_WINDOW)
            deff[sl] = jnp.where(m, dstv[sl], N_NODES + wid)
        wait_rows(b)
        fire_rows(b ^ 1)
        pltpu.sync_copy(rows, acc.at[deff], add=True)
        pltpu.sync_copy(onesv, accc.at[deff], add=True)

    load_idx(0, 0)
    fire_rows(0)

    def pair(k, carry):
        do_slot(2 * k, 0)
        do_slot(2 * k + 1, 1)
        return carry

    lax.fori_loop(0, _NSLOT // 2, pair, None)

    wait_rows(0)

    plsc.subcore_barrier()

    pltpu.sync_copy(acc.at[pl.ds(sid * _ZROWS, _ZROWS)],
                    p_out.at[pl.ds(cid * _NROWS + sid * _ZROWS, _ZROWS)])
    pltpu.sync_copy(accc.at[pl.ds(sid * _ZROWS, _ZROWS)],
                    c_out.at[pl.ds(cid * _NROWS + sid * _ZROWS, _ZROWS)])


_sc_call = functools.partial(
    pl.kernel,
    out_type=[
        jax.ShapeDtypeStruct((2 * _NROWS, D_FEAT), jnp.float32),
        jax.ShapeDtypeStruct((2 * _NROWS,), jnp.float32),
    ],
    mesh=plsc.VectorSubcoreMesh(core_axis_name="c", subcore_axis_name="s"),
    scratch_types=[
        pltpu.VMEM_SHARED((_NROWS, D_FEAT), jnp.float32),
        pltpu.VMEM_SHARED((_NROWS,), jnp.float32),
        pltpu.VMEM((_B,), jnp.int32),
        pltpu.VMEM((_B,), jnp.int32),
        pltpu.VMEM((_B,), jnp.int32),
        pltpu.VMEM((_B,), jnp.int32),
        pltpu.VMEM((_B,), jnp.int32),
        pltpu.VMEM((_B,), jnp.int32),
        pltpu.VMEM((_B,), jnp.int32),
        pltpu.VMEM((_B,), jnp.int32),
        pltpu.VMEM((_B,), jnp.int32),
        pltpu.VMEM((_B,), jnp.int32),
        pltpu.VMEM((_B, D_FEAT), jnp.float32),
        pltpu.VMEM((_B, D_FEAT), jnp.float32),
        pltpu.VMEM((_B,), jnp.float32),
        pltpu.VMEM((16, D_FEAT), jnp.float32),
        pltpu.VMEM((_ZROWS,), jnp.float32),
        pltpu.SemaphoreType.DMA,
        pltpu.SemaphoreType.DMA,
        pltpu.SemaphoreType.DMA,
        pltpu.SemaphoreType.DMA,
        pltpu.SemaphoreType.DMA,
        pltpu.SemaphoreType.DMA,
    ],
)(_sc_body)


def _combine_body(x_ref, p0_ref, p1_ref, c0_ref, c1_ref, o_ref):
    cnt = c0_ref[0, 0, :] + c1_ref[0, 0, :]
    s = p0_ref[...] + p1_ref[...]
    o_ref[...] = x_ref[...] + s / jnp.clip(cnt, 1.0, None)[:, None]


_R = 1000


def _combine(x, p0, p1, c0, c1):
    return pl.pallas_call(
        _combine_body,
        grid=(N_NODES // _R,),
        in_specs=[
            pl.BlockSpec((_R, D_FEAT), lambda i: (i, 0)),
            pl.BlockSpec((_R, D_FEAT), lambda i: (i, 0)),
            pl.BlockSpec((_R, D_FEAT), lambda i: (i, 0)),
            pl.BlockSpec((1, 1, _R), lambda i: (i, 0, 0)),
            pl.BlockSpec((1, 1, _R), lambda i: (i, 0, 0)),
        ],
        out_specs=pl.BlockSpec((_R, D_FEAT), lambda i: (i, 0)),
        out_shape=jax.ShapeDtypeStruct((N_NODES, D_FEAT), jnp.float32),
    )(x, p0, p1, c0, c1)


@jax.jit
def kernel(x, edge_index, edge_time, seed_time):
    pad = _EPAD - N_EDGES
    src = jnp.concatenate([edge_index[0], jnp.zeros((pad,), jnp.int32)])
    dst = jnp.concatenate([edge_index[1], jnp.zeros((pad,), jnp.int32)])
    et = jnp.concatenate([edge_time, jnp.full((pad,), 2 ** 30, jnp.int32)])
    pr, cr = _sc_call(x, src, dst, et, seed_time)
    p0 = pr[:N_NODES]
    p1 = pr[_NROWS:_NROWS + N_NODES]
    c0 = cr[:N_NODES].reshape(N_NODES // _R, 1, _R)
    c1 = cr[_NROWS:_NROWS + N_NODES].reshape(N_NODES // _R, 1, _R)
    return _combine(x, p0, p1, c0, c1)


# speedup vs baseline: 1.0915x; 1.0915x over previous
import functools

import jax
import jax.numpy as jnp
from jax import lax
from jax.experimental import pallas as pl
from jax.experimental.pallas import tpu as pltpu
from jax.experimental.pallas import tpu_sc as plsc

N_NODES = 10000
N_EDGES = 320000
D_FEAT = 128
TIME_WINDOW = 500

_B = 128
_TILES = 32
_NSLOT = 80
_EPAD = (_NSLOT + 1) * _TILES * _B
_NROWS = 10240
_ZROWS = _NROWS // 16


def _sc_body(x_hbm, src_hbm, dst_hbm, et_hbm, st_hbm, p_out, c_out,
             acc, accc,
             srcv0, srcv1, dstv0, dstv1, etv0, etv1, stv0, stv1,
             deff0, deff1, rows0, rows1, onesv, zb2, zb1,
             s_idx0, s_idx1, s_st0, s_st1, s_rows0, s_rows1):
    cid = lax.axis_index("c")
    sid = lax.axis_index("s")
    wid = sid * 2 + cid

    z16 = jnp.zeros((16,), jnp.float32)
    for i in range(16):
        for j in range(8):
            zb2[i, pl.ds(j * 16, 16)] = z16
    for k in range(_ZROWS // 16):
        zb1[pl.ds(k * 16, 16)] = z16
    for j in range(8):
        onesv[pl.ds(j * 16, 16)] = jnp.ones((16,), jnp.float32)

    def zloop(k, carry):
        pltpu.sync_copy(zb2, acc.at[pl.ds(sid * _ZROWS + k * 16, 16)])
        return carry

    lax.fori_loop(0, _ZROWS // 16, zloop, None)
    pltpu.sync_copy(zb1, accc.at[pl.ds(sid * _ZROWS, _ZROWS)])

    plsc.subcore_barrier()

    bufs = [
        (srcv0, dstv0, etv0, stv0, deff0, rows0, s_idx0, s_st0, s_rows0),
        (srcv1, dstv1, etv1, stv1, deff1, rows1, s_idx1, s_st1, s_rows1),
    ]

    def load_idx(g, b):
        srcv, dstv, etv = bufs[b][0], bufs[b][1], bufs[b][2]
        off = (g * _TILES + wid) * _B
        pltpu.sync_copy(src_hbm.at[pl.ds(off, _B)], srcv)
        pltpu.sync_copy(dst_hbm.at[pl.ds(off, _B)], dstv)
        pltpu.sync_copy(et_hbm.at[pl.ds(off, _B)], etv)

    def fire_rows(b):
        srcv, rows, s_rows = bufs[b][0], bufs[b][5], bufs[b][8]
        pltpu.make_async_copy(x_hbm.at[srcv], rows, s_rows).start()

    def wait_rows(b):
        srcv, rows, s_rows = bufs[b][0], bufs[b][5], bufs[b][8]
        pltpu.make_async_copy(x_hbm.at[srcv], rows, s_rows).wait()

    def fire_rows(b):
        srcv, rows, s_rows = bufs[b][0], bufs[b][5], bufs[b][8]
        pltpu.make_async_copy(x_hbm.at[srcv], rows, s_rows).start()

    def wait_rows(b):
        srcv, rows, s_rows = bufs[b][0], bufs[b][5], bufs[b][8]
        pltpu.make_async_copy(x_hbm.at[srcv], rows, s_rows).wait()

    def do_slot(g, b):
        srcv, dstv, etv, stv, deff, rows, s_idx, s_st, s_rows = bufs[b]
        load_idx(g + 1, b ^ 1)
        pltpu.async_copy(st_hbm.at[dstv], stv, s_st).wait()
        for j in range(_B // 16):
            sl = pl.ds(j * 16, 16)
            et = etv[sl]
            st = stv[sl]
            m = (et <= st) & (et > st - TIME_WINDOW)
            deff[sl] = jnp.where(m, dstv[sl], N_NODES + wid)
        pltpu.async_copy(x_hbm.at[srcv], rows, s_rows).wait()
        pltpu.sync_copy(rows, acc.at[deff], add=True)
        pltpu.sync_copy(onesv, accc.at[deff], add=True)

    load_idx(0, 0)

    def pair(k, carry):
        do_slot(2 * k, 0)
        do_slot(2 * k + 1, 1)
        return carry

    lax.fori_loop(0, _NSLOT // 2, pair, None)

    plsc.subcore_barrier()

    pltpu.sync_copy(acc.at[pl.ds(sid * _ZROWS, _ZROWS)],
                    p_out.at[pl.ds(cid * _NROWS + sid * _ZROWS, _ZROWS)])
    pltpu.sync_copy(accc.at[pl.ds(sid * _ZROWS, _ZROWS)],
                    c_out.at[pl.ds(cid * _NROWS + sid * _ZROWS, _ZROWS)])


_sc_call = functools.partial(
    pl.kernel,
    out_type=[
        jax.ShapeDtypeStruct((2 * _NROWS, D_FEAT), jnp.float32),
        jax.ShapeDtypeStruct((2 * _NROWS,), jnp.float32),
    ],
    mesh=plsc.VectorSubcoreMesh(core_axis_name="c", subcore_axis_name="s"),
    scratch_types=[
        pltpu.VMEM_SHARED((_NROWS, D_FEAT), jnp.float32),
        pltpu.VMEM_SHARED((_NROWS,), jnp.float32),
        pltpu.VMEM((_B,), jnp.int32),
        pltpu.VMEM((_B,), jnp.int32),
        pltpu.VMEM((_B,), jnp.int32),
        pltpu.VMEM((_B,), jnp.int32),
        pltpu.VMEM((_B,), jnp.int32),
        pltpu.VMEM((_B,), jnp.int32),
        pltpu.VMEM((_B,), jnp.int32),
        pltpu.VMEM((_B,), jnp.int32),
        pltpu.VMEM((_B,), jnp.int32),
        pltpu.VMEM((_B,), jnp.int32),
        pltpu.VMEM((_B, D_FEAT), jnp.float32),
        pltpu.VMEM((_B, D_FEAT), jnp.float32),
        pltpu.VMEM((_B,), jnp.float32),
        pltpu.VMEM((16, D_FEAT), jnp.float32),
        pltpu.VMEM((_ZROWS,), jnp.float32),
        pltpu.SemaphoreType.DMA,
        pltpu.SemaphoreType.DMA,
        pltpu.SemaphoreType.DMA,
        pltpu.SemaphoreType.DMA,
        pltpu.SemaphoreType.DMA,
        pltpu.SemaphoreType.DMA,
    ],
)(_sc_body)


def _combine_body(x_ref, p0_ref, p1_ref, c0_ref, c1_ref, o_ref):
    cnt = c0_ref[0, 0, :] + c1_ref[0, 0, :]
    s = p0_ref[...] + p1_ref[...]
    o_ref[...] = x_ref[...] + s / jnp.clip(cnt, 1.0, None)[:, None]


_R = 1000


def _combine(x, p0, p1, c0, c1):
    return pl.pallas_call(
        _combine_body,
        grid=(N_NODES // _R,),
        in_specs=[
            pl.BlockSpec((_R, D_FEAT), lambda i: (i, 0)),
            pl.BlockSpec((_R, D_FEAT), lambda i: (i, 0)),
            pl.BlockSpec((_R, D_FEAT), lambda i: (i, 0)),
            pl.BlockSpec((1, 1, _R), lambda i: (i, 0, 0)),
            pl.BlockSpec((1, 1, _R), lambda i: (i, 0, 0)),
        ],
        out_specs=pl.BlockSpec((_R, D_FEAT), lambda i: (i, 0)),
        out_shape=jax.ShapeDtypeStruct((N_NODES, D_FEAT), jnp.float32),
    )(x, p0, p1, c0, c1)


@jax.jit
def kernel(x, edge_index, edge_time, seed_time):
    pad = _EPAD - N_EDGES
    src = jnp.concatenate([edge_index[0], jnp.zeros((pad,), jnp.int32)])
    dst = jnp.concatenate([edge_index[1], jnp.zeros((pad,), jnp.int32)])
    et = jnp.concatenate([edge_time, jnp.full((pad,), 2 ** 30, jnp.int32)])
    pr, cr = _sc_call(x, src, dst, et, seed_time)
    p0 = pr[:N_NODES]
    p1 = pr[_NROWS:_NROWS + N_NODES]
    c0 = cr[:N_NODES].reshape(N_NODES // _R, 1, _R)
    c1 = cr[_NROWS:_NROWS + N_NODES].reshape(N_NODES // _R, 1, _R)
    return _combine(x, p0, p1, c0, c1)


# speedup vs baseline: 1.8392x; 1.6850x over previous
import functools

import jax
import jax.numpy as jnp
from jax import lax
from jax.experimental import pallas as pl
from jax.experimental.pallas import tpu as pltpu
from jax.experimental.pallas import tpu_sc as plsc

N_NODES = 10000
N_EDGES = 320000
D_FEAT = 128
TIME_WINDOW = 500

_B = 128
_NCHUNK = N_EDGES // _B
_TILES = 32
_GMAX = -(-_NCHUNK // _TILES)
_NROWS = 10240
_ZROWS = _NROWS // 16


def _sc_body(x_hbm, src_hbm, dst_hbm, et_hbm, st_hbm, p_out, c_out,
             acc, accc, srcv, dstv, etv, stv, deff, rows, onesv, zb2, zb1,
             sem):
    cid = lax.axis_index("c")
    sid = lax.axis_index("s")
    wid = sid * 2 + cid

    z16 = jnp.zeros((16,), jnp.float32)
    for i in range(16):
        for j in range(8):
            zb2[i, pl.ds(j * 16, 16)] = z16
    for k in range(_ZROWS // 16):
        zb1[pl.ds(k * 16, 16)] = z16
    for j in range(8):
        onesv[pl.ds(j * 16, 16)] = jnp.ones((16,), jnp.float32)

    def zloop(k, carry):
        pltpu.sync_copy(zb2, acc.at[pl.ds(sid * _ZROWS + k * 16, 16)])
        return carry

    lax.fori_loop(0, _ZROWS // 16, zloop, None)
    pltpu.sync_copy(zb1, accc.at[pl.ds(sid * _ZROWS, _ZROWS)])

    plsc.subcore_barrier()

    def chunk(g, carry):
        c = g * _TILES + wid

        @pl.when(c < _NCHUNK)
        def _():
            off = c * _B
            pltpu.sync_copy(src_hbm.at[pl.ds(off, _B)], srcv)
            pltpu.sync_copy(dst_hbm.at[pl.ds(off, _B)], dstv)
            pltpu.sync_copy(et_hbm.at[pl.ds(off, _B)], etv)
            pltpu.async_copy(st_hbm.at[dstv], stv, sem).wait()
            for j in range(_B // 16):
                sl = pl.ds(j * 16, 16)
                et = etv[sl]
                st = stv[sl]
                m = (et <= st) & (et > st - TIME_WINDOW)
                deff[sl] = jnp.where(m, dstv[sl], N_NODES + wid)
            pltpu.async_copy(x_hbm.at[srcv], rows, sem).wait()
            pltpu.sync_copy(rows, acc.at[deff], add=True)
            pltpu.sync_copy(onesv, accc.at[deff], add=True)

        return carry

    lax.fori_loop(0, _GMAX, chunk, None)

    plsc.subcore_barrier()

    pltpu.sync_copy(acc.at[pl.ds(sid * _ZROWS, _ZROWS)],
                    p_out.at[pl.ds(cid * _NROWS + sid * _ZROWS, _ZROWS)])
    pltpu.sync_copy(accc.at[pl.ds(sid * _ZROWS, _ZROWS)],
                    c_out.at[pl.ds(cid * _NROWS + sid * _ZROWS, _ZROWS)])


_sc_call = functools.partial(
    pl.kernel,
    out_type=[
        jax.ShapeDtypeStruct((2 * _NROWS, D_FEAT), jnp.float32),
        jax.ShapeDtypeStruct((2 * _NROWS,), jnp.float32),
    ],
    mesh=plsc.VectorSubcoreMesh(core_axis_name="c", subcore_axis_name="s"),
    scratch_types=[
        pltpu.VMEM_SHARED((_NROWS, D_FEAT), jnp.float32),
        pltpu.VMEM_SHARED((_NROWS,), jnp.float32),
        pltpu.VMEM((_B,), jnp.int32),
        pltpu.VMEM((_B,), jnp.int32),
        pltpu.VMEM((_B,), jnp.int32),
        pltpu.VMEM((_B,), jnp.int32),
        pltpu.VMEM((_B,), jnp.int32),
        pltpu.VMEM((_B, D_FEAT), jnp.float32),
        pltpu.VMEM((_B,), jnp.float32),
        pltpu.VMEM((16, D_FEAT), jnp.float32),
        pltpu.VMEM((_ZROWS,), jnp.float32),
        pltpu.SemaphoreType.DMA,
    ],
)(_sc_body)


def _combine_body(x_ref, p0_ref, p1_ref, c0_ref, c1_ref, o_ref):
    cnt = c0_ref[0, 0, :] + c1_ref[0, 0, :]
    s = p0_ref[...] + p1_ref[...]
    o_ref[...] = x_ref[...] + s / jnp.clip(cnt, 1.0, None)[:, None]


_R = 1000


def _combine(x, p0, p1, c0, c1):
    return pl.pallas_call(
        _combine_body,
        grid=(N_NODES // _R,),
        in_specs=[
            pl.BlockSpec((_R, D_FEAT), lambda i: (i, 0)),
            pl.BlockSpec((_R, D_FEAT), lambda i: (i, 0)),
            pl.BlockSpec((_R, D_FEAT), lambda i: (i, 0)),
            pl.BlockSpec((1, 1, _R), lambda i: (i, 0, 0)),
            pl.BlockSpec((1, 1, _R), lambda i: (i, 0, 0)),
        ],
        out_specs=pl.BlockSpec((_R, D_FEAT), lambda i: (i, 0)),
        out_shape=jax.ShapeDtypeStruct((N_NODES, D_FEAT), jnp.float32),
    )(x, p0, p1, c0, c1)


@jax.jit
def kernel(x, edge_index, edge_time, seed_time):
    src = edge_index[0]
    dst = edge_index[1]
    pr, cr = _sc_call(x, src, dst, edge_time, seed_time)
    p0 = pr[:N_NODES]
    p1 = pr[_NROWS:_NROWS + N_NODES]
    c0 = cr[:N_NODES].reshape(N_NODES // _R, 1, _R)
    c1 = cr[_NROWS:_NROWS + N_NODES].reshape(N_NODES // _R, 1, _R)
    return _combine(x, p0, p1, c0, c1)
